# baseline (speedup 1.0000x reference)
import functools
import math

import jax
import jax.numpy as jnp
import numpy as np
from jax import lax
from jax.experimental import pallas as pl
from jax.experimental.pallas import tpu as pltpu
from jax.experimental.pallas import tpu_sc as plsc

D_MODEL = 64
CHUNK = 32
MAX_LEN = 512

_NUM_CORES = 2
_NUM_SUBCORES = 16
_NUM_WORKERS = _NUM_CORES * _NUM_SUBCORES
_LANES = 16

_GATHER_ROWS = 128

_PREP_BLOCK = 512


def _pe_chunk_mean(d_model: int, max_len: int, chunk: int) -> np.ndarray:
    position = np.arange(max_len, dtype=np.float32)[:, None]
    div_term = np.exp(
        np.arange(0, d_model, 2, dtype=np.float32) * (-math.log(10000.0) / d_model)
    )
    pe = np.zeros((max_len, d_model), dtype=np.float32)
    pe[:, 0::2] = np.sin(position * div_term)
    pe[:, 1::2] = np.cos(position * div_term)
    n_chunks = max_len // chunk
    return pe[: n_chunks * chunk].reshape(n_chunks, chunk, d_model).mean(axis=1)


@functools.lru_cache(maxsize=None)
def _build_pack(vocab: int, d: int):
    n_blocks = (vocab + _PREP_BLOCK - 1) // _PREP_BLOCK
    words = d // 2
    out_rows = _PREP_BLOCK * words // 128

    def body(in_ref, out_ref):
        x = in_ref[...]
        ua = lax.bitcast_convert_type(x[:, :words], jnp.uint32)
        ub = lax.bitcast_convert_type(x[:, words:], jnp.uint32)

        def rnd(u):
            return u + jnp.uint32(0x7FFF) + ((u >> 16) & jnp.uint32(1))

        w = (rnd(ua) >> 16) | (rnd(ub) & jnp.uint32(0xFFFF0000))
        for q in range(_PREP_BLOCK // 128):
            out_ref[:, 32 * q:32 * q + 32] = lax.bitcast_convert_type(
                w[128 * q:128 * q + 128, :], jnp.float32)

    return pl.pallas_call(
        body,
        grid=(n_blocks,),
        in_specs=[pl.BlockSpec((_PREP_BLOCK, d), lambda i: (i, 0))],
        out_specs=pl.BlockSpec((out_rows, 128), lambda i: (i, 0)),
        out_shape=jax.ShapeDtypeStruct((n_blocks * out_rows, 128), jnp.float32),
    )


@functools.lru_cache(maxsize=None)
def _build_sc_call(batch: int, seq: int, table_rows: int, d: int):
    n_chunks = seq // CHUNK
    total_tokens = batch * seq
    steps = total_tokens // (_NUM_WORKERS * _GATHER_ROWS)
    rows_per_worker = batch // _NUM_WORKERS
    chunks_per_step = _GATHER_ROWS // CHUNK
    steps_per_row = seq // _GATHER_ROWS
    n_groups = d // 32
    words = d // 2
    scale = jnp.float32(math.sqrt(d) / CHUNK)
    mask_hi = jnp.uint32(0xFFFF0000)

    def body(ids_hbm, table_hbm, pe_hbm, out_hbm, idx_v, rows_v, out_v, pe_v,
             sem0, sem1):
        wid = lax.axis_index("s") * _NUM_CORES + lax.axis_index("c")
        sems = (sem0, sem1)

        pltpu.sync_copy(ids_hbm.at[pl.ds(wid * steps, steps)], idx_v)
        pltpu.sync_copy(pe_hbm, pe_v)

        def start(g, slot):
            pltpu.async_copy(table_hbm.at[idx_v.at[g]], rows_v.at[slot],
                             sems[slot])

        def wait(g, slot):
            pltpu.make_async_copy(table_hbm.at[idx_v.at[g]], rows_v.at[slot],
                                  sems[slot]).wait()

        def reduce(g, slot):
            b_loc = g // steps_per_row
            pe_base = (g % steps_per_row) * chunks_per_step
            for c in range(chunks_per_step):
                accs = [None] * (2 * n_groups)
                for r in range(CHUNK):
                    for h in range(n_groups):
                        w = plsc.bitcast(
                            rows_v[slot, CHUNK * c + r, pl.ds(_LANES * h, _LANES)],
                            jnp.uint32)
                        lo = plsc.bitcast(w << 16, jnp.float32)
                        hi = plsc.bitcast(w & mask_hi, jnp.float32)
                        if r == 0:
                            accs[2 * h] = lo
                            accs[2 * h + 1] = hi
                        else:
                            accs[2 * h] = accs[2 * h] + lo
                            accs[2 * h + 1] = accs[2 * h + 1] + hi
                chunk_idx = pe_base + c
                for v in range(2 * n_groups):
                    col = 16 * (v // 2) + 32 * (v % 2)
                    out_v[b_loc, chunk_idx, pl.ds(col, _LANES)] = (
                        accs[v] * scale
                        + pe_v[chunk_idx, pl.ds(col, _LANES)])

        start(0, 0)
        start(1, 1)

        def loop_body(i, carry):
            g = 2 * i
            for slot in range(2):
                gg = g + slot
                wait(gg, slot)
                reduce(gg, slot)

                @pl.when(gg + 2 < steps)
                def _():
                    start(gg + 2, slot)
            return carry

        lax.fori_loop(0, steps // 2, loop_body, 0)

        pltpu.sync_copy(
            out_v,
            out_hbm.at[pl.ds(wid * rows_per_worker, rows_per_worker)])

    return pl.kernel(
        body,
        out_type=jax.ShapeDtypeStruct((batch, n_chunks, d), jnp.float32),
        mesh=plsc.VectorSubcoreMesh(core_axis_name="c", subcore_axis_name="s"),
        compiler_params=pltpu.CompilerParams(
            use_tc_tiling_on_sc=False, needs_layout_passes=False),
        scratch_types=[
            pltpu.VMEM((steps, _GATHER_ROWS), jnp.int32),
            pltpu.VMEM((2, _GATHER_ROWS, words), jnp.float32),
            pltpu.VMEM((rows_per_worker, n_chunks, d), jnp.float32),
            pltpu.VMEM((n_chunks, d), jnp.float32),
            pltpu.SemaphoreType.DMA,
            pltpu.SemaphoreType.DMA,
        ],
    )


def kernel(token_ids, embedding):
    batch, seq = token_ids.shape
    vocab, d = embedding.shape
    t = token_ids.astype(jnp.int32)
    ids = ((t >> 9) << 9) + ((t & 127) << 2) + ((t >> 7) & 3)
    ids = ids.reshape(-1, _GATHER_ROWS)
    packed = _build_pack(vocab, d)(embedding)
    table = packed.reshape(-1, d // 2)
    pe = jnp.asarray(_pe_chunk_mean(d, seq, CHUNK))
    sc_call = _build_sc_call(batch, seq, table.shape[0], d)
    return sc_call(ids, table, pe)


# speedup vs baseline: 1.2066x; 1.2066x over previous
import functools
import math

import jax
import jax.numpy as jnp
import numpy as np
from jax import lax
from jax.experimental import pallas as pl
from jax.experimental.pallas import tpu as pltpu
from jax.experimental.pallas import tpu_sc as plsc

D_MODEL = 64
CHUNK = 32
MAX_LEN = 512

_NUM_CORES = 2
_NUM_SUBCORES = 16
_NUM_WORKERS = _NUM_CORES * _NUM_SUBCORES
_LANES = 16

_GATHER_ROWS = 128
_NBUF = 4


def _pe_chunk_mean(d_model: int, max_len: int, chunk: int) -> np.ndarray:
    position = np.arange(max_len, dtype=np.float32)[:, None]
    div_term = np.exp(
        np.arange(0, d_model, 2, dtype=np.float32) * (-math.log(10000.0) / d_model)
    )
    pe = np.zeros((max_len, d_model), dtype=np.float32)
    pe[:, 0::2] = np.sin(position * div_term)
    pe[:, 1::2] = np.cos(position * div_term)
    n_chunks = max_len // chunk
    return pe[: n_chunks * chunk].reshape(n_chunks, chunk, d_model).mean(axis=1)


@functools.lru_cache(maxsize=None)
def _build_sc_call(batch: int, seq: int, vocab: int, d: int):
    n_chunks = seq // CHUNK
    total_tokens = batch * seq
    steps = total_tokens // (_NUM_WORKERS * _GATHER_ROWS)
    rows_per_worker = batch // _NUM_WORKERS
    chunks_per_step = _GATHER_ROWS // CHUNK
    steps_per_row = seq // _GATHER_ROWS
    n_vregs = d // _LANES
    scale = jnp.float32(math.sqrt(d) / CHUNK)

    def body(ids_hbm, table_hbm, pe_hbm, out_hbm, idx_v, rows_v, out_v, pe_v,
             *sems):
        wid = lax.axis_index("s") * _NUM_CORES + lax.axis_index("c")

        pltpu.sync_copy(ids_hbm.at[pl.ds(wid * steps, steps)], idx_v)
        pltpu.sync_copy(pe_hbm, pe_v)

        def start(g, slot):
            pltpu.async_copy(table_hbm.at[idx_v.at[g]], rows_v.at[slot],
                             sems[slot])

        def wait(g, slot):
            pltpu.make_async_copy(table_hbm.at[idx_v.at[g]], rows_v.at[slot],
                                  sems[slot]).wait()

        def reduce(g, slot):
            b_loc = g // steps_per_row
            pe_base = (g % steps_per_row) * chunks_per_step
            for c in range(chunks_per_step):
                accs = [rows_v[slot, CHUNK * c, pl.ds(_LANES * v, _LANES)]
                        for v in range(n_vregs)]
                for r in range(1, CHUNK):
                    for v in range(n_vregs):
                        accs[v] = accs[v] + rows_v[
                            slot, CHUNK * c + r, pl.ds(_LANES * v, _LANES)]
                chunk_idx = pe_base + c
                for v in range(n_vregs):
                    out_v[b_loc, chunk_idx, pl.ds(_LANES * v, _LANES)] = (
                        accs[v] * scale
                        + pe_v[chunk_idx, pl.ds(_LANES * v, _LANES)])

        for p in range(_NBUF):
            start(p, p)

        def loop_body(i, carry):
            g = _NBUF * i
            for slot in range(_NBUF):
                gg = g + slot
                wait(gg, slot)
                reduce(gg, slot)

                @pl.when(gg + _NBUF < steps)
                def _():
                    start(gg + _NBUF, slot)
            return carry

        lax.fori_loop(0, steps // _NBUF, loop_body, 0)

        pltpu.sync_copy(
            out_v,
            out_hbm.at[pl.ds(wid * rows_per_worker, rows_per_worker)])

    return pl.kernel(
        body,
        out_type=jax.ShapeDtypeStruct((batch, n_chunks, d), jnp.float32),
        mesh=plsc.VectorSubcoreMesh(core_axis_name="c", subcore_axis_name="s"),
        compiler_params=pltpu.CompilerParams(use_tc_tiling_on_sc=False),
        scratch_types=[
            pltpu.VMEM((steps, _GATHER_ROWS), jnp.int32),
            pltpu.VMEM((_NBUF, _GATHER_ROWS, d), jnp.float32),
            pltpu.VMEM((rows_per_worker, n_chunks, d), jnp.float32),
            pltpu.VMEM((n_chunks, d), jnp.float32),
        ] + [pltpu.SemaphoreType.DMA] * _NBUF,
    )


def kernel(token_ids, embedding):
    batch, seq = token_ids.shape
    vocab, d = embedding.shape
    ids = token_ids.astype(jnp.int32).reshape(-1, _GATHER_ROWS)
    pe = jnp.asarray(_pe_chunk_mean(d, seq, CHUNK))
    sc_call = _build_sc_call(batch, seq, vocab, d)
    return sc_call(ids, embedding, pe)


# speedup vs baseline: 1.4176x; 1.1749x over previous
import functools
import math

import jax
import jax.numpy as jnp
import numpy as np
from jax import lax
from jax.experimental import pallas as pl
from jax.experimental.pallas import tpu as pltpu
from jax.experimental.pallas import tpu_sc as plsc

D_MODEL = 64
CHUNK = 32
MAX_LEN = 512

_NUM_CORES = 2
_NUM_SUBCORES = 16
_NUM_WORKERS = _NUM_CORES * _NUM_SUBCORES
_LANES = 16

_GATHER_ROWS = 128


def _pe_chunk_mean(d_model: int, max_len: int, chunk: int) -> np.ndarray:
    position = np.arange(max_len, dtype=np.float32)[:, None]
    div_term = np.exp(
        np.arange(0, d_model, 2, dtype=np.float32) * (-math.log(10000.0) / d_model)
    )
    pe = np.zeros((max_len, d_model), dtype=np.float32)
    pe[:, 0::2] = np.sin(position * div_term)
    pe[:, 1::2] = np.cos(position * div_term)
    n_chunks = max_len // chunk
    return pe[: n_chunks * chunk].reshape(n_chunks, chunk, d_model).mean(axis=1)


@functools.lru_cache(maxsize=None)
def _build_sc_call(batch: int, seq: int, vocab: int, d: int):
    n_chunks = seq // CHUNK
    total_tokens = batch * seq
    steps = total_tokens // (_NUM_WORKERS * _GATHER_ROWS)
    out_rows_per_worker = batch * n_chunks // _NUM_WORKERS
    chunks_per_step = _GATHER_ROWS // CHUNK
    n_vregs = d // _LANES
    scale = math.sqrt(d) / CHUNK

    def body(ids_hbm, table_hbm, pe_hbm, out_hbm, idx_v, rows_v, out_v, pe_v,
             sem0, sem1):
        wid = lax.axis_index("s") * _NUM_CORES + lax.axis_index("c")
        sems = (sem0, sem1)

        pltpu.sync_copy(ids_hbm.at[pl.ds(wid * steps, steps)], idx_v)
        pltpu.sync_copy(pe_hbm, pe_v)

        def start(g, slot):
            pltpu.async_copy(table_hbm.at[idx_v.at[g]], rows_v.at[slot],
                             sems[slot])

        def wait(g, slot):
            pltpu.make_async_copy(table_hbm.at[idx_v.at[g]], rows_v.at[slot],
                                  sems[slot]).wait()

        def reduce(g, slot):
            pe_base = (g % (n_chunks // chunks_per_step)) * chunks_per_step
            out_base = g * chunks_per_step
            for c in range(chunks_per_step):
                accs = [rows_v[slot, CHUNK * c, pl.ds(_LANES * v, _LANES)]
                        for v in range(n_vregs)]
                for r in range(1, CHUNK):
                    for v in range(n_vregs):
                        accs[v] = accs[v] + rows_v[
                            slot, CHUNK * c + r, pl.ds(_LANES * v, _LANES)]
                for v in range(n_vregs):
                    out_v[out_base + c, pl.ds(_LANES * v, _LANES)] = (
                        accs[v] * scale
                        + pe_v[pe_base + c, pl.ds(_LANES * v, _LANES)])

        start(0, 0)
        start(1, 1)

        def loop_body(i, carry):
            g = 2 * i
            for slot in range(2):
                gg = g + slot
                wait(gg, slot)
                reduce(gg, slot)

                @pl.when(gg + 2 < steps)
                def _():
                    start(gg + 2, slot)
            return carry

        lax.fori_loop(0, steps // 2, loop_body, 0)

        pltpu.sync_copy(
            out_v,
            out_hbm.at[pl.ds(wid * out_rows_per_worker, out_rows_per_worker)])

    return pl.kernel(
        body,
        out_type=jax.ShapeDtypeStruct((batch * n_chunks, d), jnp.float32),
        mesh=plsc.VectorSubcoreMesh(core_axis_name="c", subcore_axis_name="s"),
        compiler_params=pltpu.CompilerParams(use_tc_tiling_on_sc=False),
        scratch_types=[
            pltpu.VMEM((steps, _GATHER_ROWS), jnp.int32),
            pltpu.VMEM((2, _GATHER_ROWS, d), jnp.float32),
            pltpu.VMEM((out_rows_per_worker, d), jnp.float32),
            pltpu.VMEM((n_chunks, d), jnp.float32),
            pltpu.SemaphoreType.DMA,
            pltpu.SemaphoreType.DMA,
        ],
    )


def kernel(token_ids, embedding):
    batch, seq = token_ids.shape
    vocab, d = embedding.shape
    n_chunks = seq // CHUNK
    ids = token_ids.astype(jnp.int32).reshape(-1, _GATHER_ROWS)
    pe_mean = jnp.asarray(_pe_chunk_mean(d, seq, CHUNK))
    sc_call = _build_sc_call(batch, seq, vocab, d)
    out = sc_call(ids, embedding, pe_mean)
    return out.reshape(batch, n_chunks, d)
